# baseline (speedup 1.0000x reference)
import functools

import jax
import jax.numpy as jnp
from jax import lax
from jax.experimental import pallas as pl
from jax.experimental.pallas import tpu as pltpu
from jax.experimental.pallas import tpu_sc as plsc

B, W, P, D = 4, 1024, 8192, 64
WBLK = 256
NW_BLK = W // WBLK
PBLK = 512
NP = P // PBLK

ALPHA_C = 1.0
BETA_C = 0.5

NC, NS, L = 2, 16, 16
NW = NC * NS
RPW = (B * W) // NW
GROUPS = RPW // L


def _argmin_body(w_ref, bp_ref, idx_ref, minval, minidx):
    b = pl.program_id(0)
    p = pl.program_id(2)

    @pl.when(p == 0)
    def _init():
        minval[...] = jnp.full_like(minval[...], jnp.inf)
        minidx[...] = jnp.zeros_like(minidx[...])

    wm2 = -2.0 * w_ref[0]
    bp = bp_ref[0]
    ab = lax.dot_general(wm2, bp, (((1,), (1,)), ((), ())),
                         preferred_element_type=jnp.float32)
    bp2 = jnp.sum(bp * bp, axis=1)
    score = ab + bp2[None, :]
    bmin = jnp.min(score, axis=1, keepdims=True)
    cols = lax.broadcasted_iota(jnp.int32, score.shape, 1)
    bidx = jnp.min(jnp.where(score == bmin, cols, jnp.int32(2**31 - 1)),
                   axis=1, keepdims=True) + (p * PBLK + b * P)
    better = bmin < minval[...]
    minval[...] = jnp.where(better, bmin, minval[...])
    minidx[...] = jnp.where(better, bidx, minidx[...])

    @pl.when(p == NP - 1)
    def _emit():
        idx_ref[0] = minidx[...]


def _nn_indices(waypoints, boundarypoints):
    return pl.pallas_call(
        _argmin_body,
        grid=(B, NW_BLK, NP),
        in_specs=[
            pl.BlockSpec((1, WBLK, D), lambda b, wb, p: (b, wb, 0)),
            pl.BlockSpec((1, PBLK, D), lambda b, wb, p: (b, p, 0)),
        ],
        out_specs=pl.BlockSpec((1, WBLK, 1), lambda b, wb, p: (b, wb, 0)),
        out_shape=jax.ShapeDtypeStruct((B, W, 1), jnp.int32),
        scratch_shapes=[
            pltpu.VMEM((WBLK, 1), jnp.float32),
            pltpu.VMEM((WBLK, 1), jnp.int32),
        ],
        compiler_params=pltpu.CompilerParams(
            dimension_semantics=("arbitrary", "arbitrary", "arbitrary")),
    )(waypoints, boundarypoints)


def _sc_loss_body(way_hbm, pn_hbm, idx_hbm, out_hbm,
                  idx_v, wbuf, pnbuf, tot_v, sem1):
    wid = lax.axis_index("s") * NC + lax.axis_index("c")
    base = wid * RPW
    pltpu.sync_copy(idx_hbm.at[pl.ds(base, RPW)], idx_v)
    cp1 = pltpu.async_copy(pn_hbm.at[idx_v], pnbuf, sem1)
    pltpu.sync_copy(way_hbm.at[pl.ds(base, RPW)], wbuf)
    cp1.wait()

    lane = lax.iota(jnp.int32, 16)

    def group(g, tot):
        def row(j, dpvec):
            r = g * L + j
            acc = jnp.zeros((L,), jnp.float32)
            for c in range(D // L):
                sl = pl.ds(c * L, L)
                nsl = pl.ds(D + c * L, L)
                acc = acc + (wbuf[r, sl] - pnbuf[r, sl]) * pnbuf[r, nsl]
            dp = jnp.sum(acc)
            return jnp.where(lane == j, dp, dpvec)

        dpvec = lax.fori_loop(0, L, row, jnp.zeros((L,), jnp.float32))
        er = jnp.where(dpvec > 0.0, ALPHA_C * dpvec,
                       jnp.exp(BETA_C * dpvec) - 1.0)
        return tot + er

    tot = lax.fori_loop(0, GROUPS, group, jnp.zeros((L,), jnp.float32))
    tot_v[...] = tot
    pltpu.sync_copy(tot_v, out_hbm.at[wid])


def _sc_loss_partials(way_flat, pn_flat, idx_flat):
    mesh = plsc.VectorSubcoreMesh(core_axis_name="c", subcore_axis_name="s")
    return pl.kernel(
        _sc_loss_body,
        out_type=jax.ShapeDtypeStruct((NW, L), jnp.float32),
        mesh=mesh,
        scratch_types=[
            pltpu.VMEM((RPW,), jnp.int32),
            pltpu.VMEM((RPW, D), jnp.float32),
            pltpu.VMEM((RPW, 2 * D), jnp.float32),
            pltpu.VMEM((L,), jnp.float32),
            pltpu.SemaphoreType.DMA,
        ],
        compiler_params=pltpu.CompilerParams(needs_layout_passes=False),
    )(way_flat, pn_flat, idx_flat)


def _reduce_body(part_ref, out_ref):
    out_ref[...] = jnp.sum(part_ref[...], keepdims=True) * (1.0 / (B * W))


def _reduce_mean(partials):
    return pl.pallas_call(
        _reduce_body,
        out_shape=jax.ShapeDtypeStruct((1, 1), jnp.float32),
    )(partials)


def kernel(waypoints, boundarypoints, boundarynormals):
    idx = _nn_indices(waypoints, boundarypoints)
    idx_flat = idx.reshape(B * W)
    way_flat = waypoints.reshape(B * W, D)
    pn_flat = jnp.concatenate(
        [boundarypoints.reshape(B * P, D), boundarynormals.reshape(B * P, D)],
        axis=1)
    partials = _sc_loss_partials(way_flat, pn_flat, idx_flat)
    return _reduce_mean(partials)[0, 0]


# speedup vs baseline: 2.1592x; 2.1592x over previous
import functools

import jax
import jax.numpy as jnp
from jax import lax
from jax.experimental import pallas as pl
from jax.experimental.pallas import tpu as pltpu
from jax.experimental.pallas import tpu_sc as plsc

B, W, P, D = 4, 1024, 8192, 64
WBLK = 512
NW_BLK = W // WBLK
PBLK = 1024
NP = P // PBLK
LANES = 128
NT = PBLK // LANES
NTILES = P // LANES

ALPHA_C = 1.0
BETA_C = 0.5

NC, NS, L = 2, 16, 16
NW = NC * NS
RPW = (B * W) // NW
GROUPS = RPW // L


def _argmin_body(w_ref, bp_ref, bn_ref, idx_ref, pn_ref, rmin, ridx, bp2_s):
    b = pl.program_id(0)
    p = pl.program_id(1)
    wb = pl.program_id(2)
    wsl = pl.ds(wb * WBLK, WBLK)

    @pl.when(p == 0)
    def _init():
        rmin[wsl, :] = jnp.full((WBLK, LANES), jnp.inf, jnp.float32)
        ridx[wsl, :] = jnp.zeros((WBLK, LANES), jnp.float32)

    @pl.when(jnp.logical_and(p == 0, wb == 0))
    def _bp2():
        for tt in range(NTILES):
            blk = bp_ref[0, tt * LANES:(tt + 1) * LANES, :]
            bp2_s[tt, :] = jnp.sum(blk * blk, axis=1)

    bp = bp_ref[0, pl.ds(p * PBLK, PBLK), :]
    wm2 = -2.0 * w_ref[0, wsl, :]
    ab = lax.dot_general(wm2, bp, (((1,), (1,)), ((), ())),
                         preferred_element_type=jnp.float32)

    rm = rmin[wsl, :]
    ri = ridx[wsl, :]
    for t in range(NT):
        lo, hi = t * LANES, (t + 1) * LANES
        tile = ab[:, lo:hi] + bp2_s[pl.ds(p * NT + t, 1), :]
        tid = (p * NT + t).astype(jnp.float32)
        better = tile < rm
        rm = jnp.where(better, tile, rm)
        ri = jnp.where(better, tid, ri)
    rmin[wsl, :] = rm
    ridx[wsl, :] = ri

    pn_ref[0] = jnp.concatenate([bp, bn_ref[0, pl.ds(p * PBLK, PBLK), :]],
                                axis=1)

    @pl.when(p == NP - 1)
    def _emit():
        lanef = lax.broadcasted_iota(
            jnp.int32, (WBLK, LANES), 1).astype(jnp.float32)
        colf = ri * jnp.float32(LANES) + lanef
        m = jnp.min(rm, axis=1, keepdims=True)
        csel = jnp.where(rm == m, colf, jnp.float32(3e38))
        cmin = jnp.min(csel, axis=1, keepdims=True)
        idx_ref[0] = cmin.astype(jnp.int32) + b * P


def _nn_indices(waypoints, boundarypoints, boundarynormals):
    return pl.pallas_call(
        _argmin_body,
        grid=(B, NP, NW_BLK),
        in_specs=[
            pl.BlockSpec((1, W, D), lambda b, p, wb: (b, 0, 0)),
            pl.BlockSpec((1, P, D), lambda b, p, wb: (b, 0, 0)),
            pl.BlockSpec((1, P, D), lambda b, p, wb: (b, 0, 0)),
        ],
        out_specs=[
            pl.BlockSpec((1, WBLK, 1), lambda b, p, wb: (b, wb, 0)),
            pl.BlockSpec((1, PBLK, 2 * D), lambda b, p, wb: (b, p, 0)),
        ],
        out_shape=[
            jax.ShapeDtypeStruct((B, W, 1), jnp.int32),
            jax.ShapeDtypeStruct((B, P, 2 * D), jnp.float32),
        ],
        scratch_shapes=[
            pltpu.VMEM((W, LANES), jnp.float32),
            pltpu.VMEM((W, LANES), jnp.float32),
            pltpu.VMEM((NTILES, LANES), jnp.float32),
        ],
        compiler_params=pltpu.CompilerParams(
            dimension_semantics=("arbitrary", "arbitrary", "arbitrary")),
    )(waypoints, boundarypoints, boundarynormals)


def _sc_loss_body(way_hbm, pn_hbm, idx_hbm, out_hbm,
                  idx_v, wbuf, pnbuf, tot_v, sem1):
    wid = lax.axis_index("s") * NC + lax.axis_index("c")
    base = wid * RPW
    pltpu.sync_copy(idx_hbm.at[pl.ds(base, RPW)], idx_v)
    cp1 = pltpu.async_copy(pn_hbm.at[idx_v], pnbuf, sem1)
    pltpu.sync_copy(way_hbm.at[pl.ds(base, RPW)], wbuf)
    cp1.wait()

    lane = lax.iota(jnp.int32, 16)

    def group(g, tot):
        def row(j, dpvec):
            r = g * L + j
            acc = jnp.zeros((L,), jnp.float32)
            for c in range(D // L):
                sl = pl.ds(c * L, L)
                nsl = pl.ds(D + c * L, L)
                acc = acc + (wbuf[r, sl] - pnbuf[r, sl]) * pnbuf[r, nsl]
            dp = jnp.sum(acc)
            return jnp.where(lane == j, dp, dpvec)

        dpvec = lax.fori_loop(0, L, row, jnp.zeros((L,), jnp.float32))
        er = jnp.where(dpvec > 0.0, ALPHA_C * dpvec,
                       jnp.exp(BETA_C * dpvec) - 1.0)
        return tot + er

    tot = lax.fori_loop(0, GROUPS, group, jnp.zeros((L,), jnp.float32))
    tot_v[...] = tot
    pltpu.sync_copy(tot_v, out_hbm.at[wid])


def _sc_loss_partials(way_flat, pn_flat, idx_flat):
    mesh = plsc.VectorSubcoreMesh(core_axis_name="c", subcore_axis_name="s")
    return pl.kernel(
        _sc_loss_body,
        out_type=jax.ShapeDtypeStruct((NW, L), jnp.float32),
        mesh=mesh,
        scratch_types=[
            pltpu.VMEM((RPW,), jnp.int32),
            pltpu.VMEM((RPW, D), jnp.float32),
            pltpu.VMEM((RPW, 2 * D), jnp.float32),
            pltpu.VMEM((L,), jnp.float32),
            pltpu.SemaphoreType.DMA,
        ],
        compiler_params=pltpu.CompilerParams(needs_layout_passes=False),
    )(way_flat, pn_flat, idx_flat)


def _reduce_body(part_ref, out_ref):
    out_ref[...] = jnp.sum(part_ref[...], keepdims=True) * (1.0 / (B * W))


def _reduce_mean(partials):
    return pl.pallas_call(
        _reduce_body,
        out_shape=jax.ShapeDtypeStruct((1, 1), jnp.float32),
    )(partials)


def kernel(waypoints, boundarypoints, boundarynormals):
    idx, pn = _nn_indices(waypoints, boundarypoints, boundarynormals)
    idx_flat = idx.reshape(B * W)
    way_flat = waypoints.reshape(B * W, D)
    pn_flat = pn.reshape(B * P, 2 * D)
    partials = _sc_loss_partials(way_flat, pn_flat, idx_flat)
    return _reduce_mean(partials)[0, 0]
